# Initial kernel scaffold; baseline (speedup 1.0000x reference)
import jax
import jax.numpy as jnp
from jax.experimental import pallas as pl


def kernel(edge_index, edge_weight, x, W):
    raise NotImplementedError("write your pallas kernel here")


# speedup vs baseline: 4.7824x; 4.7824x over previous
import dataclasses
import functools

import jax
import jax.numpy as jnp
from jax import lax
from jax.experimental import pallas as pl
from jax.experimental.pallas import tpu as pltpu
from jax.experimental.pallas import tpu_sc as plsc

N = 10000
E = 320000
D = 128
NT = 16
NC = 2
NW = NC * NT
RPT = 624
TAIL = N - NT * RPT
CH = 128
NCHUNK = E // CH
ZR = 104


def _mm(x, W):
    BM = 2000

    def body(x_ref, w_ref, o_ref):
        o_ref[...] = jnp.dot(x_ref[...], w_ref[...],
                             preferred_element_type=jnp.float32,
                             precision=jax.lax.Precision.HIGHEST)

    return pl.pallas_call(
        body,
        grid=(N // BM,),
        in_specs=[pl.BlockSpec((BM, D), lambda i: (i, 0)),
                  pl.BlockSpec((D, D), lambda i: (0, 0))],
        out_specs=pl.BlockSpec((BM, D), lambda i: (i, 0)),
        out_shape=jax.ShapeDtypeStruct((N, D), jnp.float32),
    )(x, W)


def _sc_spmm(h, rows_i, cols_i, edge_weight):
    mesh = plsc.VectorSubcoreMesh(core_axis_name="c", subcore_axis_name="s")
    cparams = pltpu.CompilerParams()
    if "needs_layout_passes" in pltpu.CompilerParams.__dataclass_fields__:
        cparams = dataclasses.replace(cparams, needs_layout_passes=False)

    @functools.partial(
        pl.kernel,
        out_type=jax.ShapeDtypeStruct((NC, N, D), jnp.float32),
        mesh=mesh,
        compiler_params=cparams,
        scratch_types=[
            pltpu.VMEM_SHARED((N, D), jnp.float32),
            pltpu.VMEM((CH, D), jnp.float32),
            pltpu.VMEM((CH,), jnp.int32),
            pltpu.VMEM((CH,), jnp.int32),
            pltpu.VMEM((CH,), jnp.float32),
            pltpu.VMEM((ZR, D), jnp.float32),
            pltpu.SemaphoreType.DMA,
        ],
    )
    def k(h_hbm, er_hbm, ec_hbm, ew_hbm, out_hbm, acc, gbuf, cbuf, rbuf,
          wbuf, zbuf, sem):
        c = lax.axis_index("c")
        s = lax.axis_index("s")
        base = s * RPT
        last = s == NT - 1

        @pl.loop(0, ZR)
        def _(r):
            for q in range(D // 16):
                zbuf[r, pl.ds(16 * q, 16)] = jnp.zeros((16,), jnp.float32)

        for kk in range(RPT // ZR):
            pltpu.sync_copy(zbuf, acc.at[pl.ds(base + kk * ZR, ZR)])

        @pl.when(last)
        def _():
            pltpu.sync_copy(zbuf.at[pl.ds(0, TAIL)],
                            acc.at[pl.ds(NT * RPT, TAIL)])

        plsc.subcore_barrier()

        w = c * NT + s
        n = (NCHUNK - w + NW - 1) // NW

        def chunk_body(i, carry):
            e0 = (w + i * NW) * CH
            pltpu.sync_copy(ec_hbm.at[pl.ds(e0, CH)], cbuf)
            pltpu.sync_copy(er_hbm.at[pl.ds(e0, CH)], rbuf)
            pltpu.sync_copy(ew_hbm.at[pl.ds(e0, CH)], wbuf)
            pltpu.async_copy(h_hbm.at[cbuf], gbuf, sem).wait()

            @pl.loop(0, CH)
            def _(e):
                wv = plsc.load_gather(wbuf, [jnp.full((16,), e, jnp.int32)])
                for q in range(D // 16):
                    sl = pl.ds(16 * q, 16)
                    gbuf[e, sl] = gbuf[e, sl] * wv

            pltpu.sync_copy(gbuf, acc.at[rbuf], add=True)
            return carry

        lax.fori_loop(0, n, chunk_body, 0)
        plsc.subcore_barrier()
        pltpu.sync_copy(acc.at[pl.ds(base, RPT)],
                        out_hbm.at[c, pl.ds(base, RPT)])

        @pl.when(last)
        def _():
            pltpu.sync_copy(acc.at[pl.ds(NT * RPT, TAIL)],
                            out_hbm.at[c, pl.ds(NT * RPT, TAIL)])

    return k(h, rows_i, cols_i, edge_weight)


def _combine_relu(p):
    BM = 2000

    def body(p_ref, o_ref):
        o_ref[...] = jnp.maximum(p_ref[0] + p_ref[1], 0.0)

    return pl.pallas_call(
        body,
        grid=(N // BM,),
        in_specs=[pl.BlockSpec((NC, BM, D), lambda i: (0, i, 0))],
        out_specs=pl.BlockSpec((BM, D), lambda i: (i, 0)),
        out_shape=jax.ShapeDtypeStruct((N, D), jnp.float32),
    )(p)


def kernel(edge_index, edge_weight, x, W):
    h = _mm(x, W)
    p = _sc_spmm(h, edge_index[0], edge_index[1], edge_weight)
    return _combine_relu(p)


# speedup vs baseline: 10.0233x; 2.0959x over previous
import dataclasses
import functools

import jax
import jax.numpy as jnp
from jax import lax
from jax.experimental import pallas as pl
from jax.experimental.pallas import tpu as pltpu
from jax.experimental.pallas import tpu_sc as plsc

N = 10000
E = 320000
D = 128
NT = 16
NC = 2
NW = NC * NT
RPT = 624
TAIL = N - NT * RPT
CH = 128
NCHUNK = E // CH
ZR = 104


def _mm(x, W):
    BM = 2000

    def body(x_ref, w_ref, o_ref):
        o_ref[...] = jnp.dot(x_ref[...], w_ref[...],
                             preferred_element_type=jnp.float32,
                             precision=jax.lax.Precision.HIGHEST)

    return pl.pallas_call(
        body,
        grid=(N // BM,),
        in_specs=[pl.BlockSpec((BM, D), lambda i: (i, 0)),
                  pl.BlockSpec((D, D), lambda i: (0, 0))],
        out_specs=pl.BlockSpec((BM, D), lambda i: (i, 0)),
        out_shape=jax.ShapeDtypeStruct((N, D), jnp.float32),
    )(x, W)


def _sc_spmm(h, rows_i, cols_i, edge_weight):
    mesh = plsc.VectorSubcoreMesh(core_axis_name="c", subcore_axis_name="s")
    cparams = pltpu.CompilerParams()
    if "needs_layout_passes" in pltpu.CompilerParams.__dataclass_fields__:
        cparams = dataclasses.replace(cparams, needs_layout_passes=False)

    @functools.partial(
        pl.kernel,
        out_type=jax.ShapeDtypeStruct((NC, N, D), jnp.float32),
        mesh=mesh,
        compiler_params=cparams,
        scratch_types=[
            pltpu.VMEM_SHARED((N, D), jnp.float32),
            pltpu.VMEM((CH, D), jnp.float32),
            pltpu.VMEM((CH, D), jnp.float32),
            pltpu.VMEM((CH,), jnp.int32),
            pltpu.VMEM((CH,), jnp.int32),
            pltpu.VMEM((CH,), jnp.int32),
            pltpu.VMEM((CH,), jnp.int32),
            pltpu.VMEM((CH,), jnp.float32),
            pltpu.VMEM((CH,), jnp.float32),
            pltpu.VMEM((ZR, D), jnp.float32),
            pltpu.SemaphoreType.DMA,
            pltpu.SemaphoreType.DMA,
            pltpu.SemaphoreType.DMA,
            pltpu.SemaphoreType.DMA,
        ],
    )
    def k(h_hbm, er_hbm, ec_hbm, ew_hbm, out_hbm, acc, gb0, gb1, cb0, cb1,
          rb0, rb1, wb0, wb1, zbuf, gsem0, gsem1, isem0, isem1):
        c = lax.axis_index("c")
        s = lax.axis_index("s")
        base = s * RPT
        last = s == NT - 1

        gb = (gb0, gb1)
        cb = (cb0, cb1)
        rb = (rb0, rb1)
        wb = (wb0, wb1)
        gsem = (gsem0, gsem1)
        isem = (isem0, isem1)

        @pl.loop(0, ZR)
        def _(r):
            for q in range(D // 16):
                zbuf[r, pl.ds(16 * q, 16)] = jnp.zeros((16,), jnp.float32)

        for kk in range(RPT // ZR):
            pltpu.sync_copy(zbuf, acc.at[pl.ds(base + kk * ZR, ZR)])

        @pl.when(last)
        def _():
            pltpu.sync_copy(zbuf.at[pl.ds(0, TAIL)],
                            acc.at[pl.ds(NT * RPT, TAIL)])

        plsc.subcore_barrier()

        w = c * NT + s
        n = (NCHUNK - w + NW - 1) // NW

        def start_idx(i, p):
            e0 = (w + i * NW) * CH
            pltpu.async_copy(ec_hbm.at[pl.ds(e0, CH)], cb[p], isem[p])
            pltpu.async_copy(er_hbm.at[pl.ds(e0, CH)], rb[p], isem[p])
            pltpu.async_copy(ew_hbm.at[pl.ds(e0, CH)], wb[p], isem[p])

        def wait_idx(p):
            pltpu.make_async_copy(ec_hbm.at[pl.ds(0, CH)], cb[p], isem[p]).wait()
            pltpu.make_async_copy(er_hbm.at[pl.ds(0, CH)], rb[p], isem[p]).wait()
            pltpu.make_async_copy(ew_hbm.at[pl.ds(0, CH)], wb[p], isem[p]).wait()

        def process(j, p):
            q = 1 - p

            @pl.when(j + 1 < n)
            def _():
                wait_idx(q)
                pltpu.async_copy(h_hbm.at[cb[q]], gb[q], gsem[q])

            pltpu.make_async_copy(h_hbm.at[pl.ds(0, CH)], gb[p], gsem[p]).wait()

            @pl.loop(0, CH, step=16)
            def _(b):
                wvec = wb[p][pl.ds(pl.multiple_of(b, 16), 16)]
                for i in range(16):
                    wv = lax.gather(
                        wvec, jnp.full((16, 1), i, jnp.int32),
                        lax.GatherDimensionNumbers(
                            offset_dims=(), collapsed_slice_dims=(0,),
                            start_index_map=(0,)),
                        slice_sizes=(1,),
                        mode=lax.GatherScatterMode.PROMISE_IN_BOUNDS)
                    for qq in range(D // 16):
                        sl = pl.ds(16 * qq, 16)
                        gb[p][b + i, sl] = gb[p][b + i, sl] * wv

            pltpu.sync_copy(gb[p], acc.at[rb[p]], add=True)

            @pl.when(j + 2 < n)
            def _():
                start_idx(j + 2, p)

        start_idx(0, 0)
        wait_idx(0)
        pltpu.async_copy(h_hbm.at[cb[0]], gb[0], gsem[0])
        start_idx(1, 1)

        def chunk_body(j, carry):
            @pl.when(j % 2 == 0)
            def _():
                process(j, 0)

            @pl.when(j % 2 == 1)
            def _():
                process(j, 1)

            return carry

        lax.fori_loop(0, n, chunk_body, 0)
        plsc.subcore_barrier()
        pltpu.sync_copy(acc.at[pl.ds(base, RPT)],
                        out_hbm.at[c, pl.ds(base, RPT)])

        @pl.when(last)
        def _():
            pltpu.sync_copy(acc.at[pl.ds(NT * RPT, TAIL)],
                            out_hbm.at[c, pl.ds(NT * RPT, TAIL)])

    return k(h, rows_i, cols_i, edge_weight)


def _combine_relu(p):
    BM = 2000

    def body(p_ref, o_ref):
        o_ref[...] = jnp.maximum(p_ref[0] + p_ref[1], 0.0)

    return pl.pallas_call(
        body,
        grid=(N // BM,),
        in_specs=[pl.BlockSpec((NC, BM, D), lambda i: (0, i, 0))],
        out_specs=pl.BlockSpec((BM, D), lambda i: (i, 0)),
        out_shape=jax.ShapeDtypeStruct((N, D), jnp.float32),
    )(p)


def kernel(edge_index, edge_weight, x, W):
    h = _mm(x, W)
    p = _sc_spmm(h, edge_index[0], edge_index[1], edge_weight)
    return _combine_relu(p)


# speedup vs baseline: 11.6714x; 1.1644x over previous
import dataclasses
import functools

import jax
import jax.numpy as jnp
from jax import lax
from jax.experimental import pallas as pl
from jax.experimental.pallas import tpu as pltpu
from jax.experimental.pallas import tpu_sc as plsc

N = 10000
E = 320000
D = 128
NT = 16
NC = 2
NW = NC * NT
RPT = 624
TAIL = N - NT * RPT
CH = 128
NCHUNK = E // CH
ZR = 104


def _mm(x, W):
    BM = 2000

    def body(x_ref, w_ref, o_ref):
        o_ref[...] = jnp.dot(x_ref[...], w_ref[...],
                             preferred_element_type=jnp.float32,
                             precision=jax.lax.Precision.HIGHEST)

    return pl.pallas_call(
        body,
        grid=(N // BM,),
        in_specs=[pl.BlockSpec((BM, D), lambda i: (i, 0)),
                  pl.BlockSpec((D, D), lambda i: (0, 0))],
        out_specs=pl.BlockSpec((BM, D), lambda i: (i, 0)),
        out_shape=jax.ShapeDtypeStruct((N, D), jnp.float32),
    )(x, W)


def _sc_spmm(h, rows_i, cols_i, edge_weight):
    mesh = plsc.VectorSubcoreMesh(core_axis_name="c", subcore_axis_name="s")
    cparams = pltpu.CompilerParams()
    if "needs_layout_passes" in pltpu.CompilerParams.__dataclass_fields__:
        cparams = dataclasses.replace(cparams, needs_layout_passes=False)

    @functools.partial(
        pl.kernel,
        out_type=jax.ShapeDtypeStruct((NC, N, D), jnp.float32),
        mesh=mesh,
        compiler_params=cparams,
        scratch_types=(
            [pltpu.VMEM_SHARED((N, D), jnp.float32)]
            + [pltpu.VMEM((CH, D), jnp.float32)] * 2
            + [pltpu.VMEM((CH,), jnp.int32)] * 4
            + [pltpu.VMEM((CH,), jnp.int32)] * 4
            + [pltpu.VMEM((CH,), jnp.float32)] * 4
            + [pltpu.VMEM((ZR, D), jnp.float32)]
            + [pltpu.SemaphoreType.DMA] * 2
            + [pltpu.SemaphoreType.DMA] * 4
            + [pltpu.SemaphoreType.DMA] * 2
        ),
    )
    def k(h_hbm, er_hbm, ec_hbm, ew_hbm, out_hbm, acc, gb0, gb1,
          cb0, cb1, cb2, cb3, rb0, rb1, rb2, rb3, wb0, wb1, wb2, wb3,
          zbuf, gsem0, gsem1, isem0, isem1, isem2, isem3, ssem0, ssem1):
        c = lax.axis_index("c")
        s = lax.axis_index("s")
        base = s * RPT
        last = s == NT - 1

        gb = (gb0, gb1)
        cb = (cb0, cb1, cb2, cb3)
        rb = (rb0, rb1, rb2, rb3)
        wb = (wb0, wb1, wb2, wb3)
        gsem = (gsem0, gsem1)
        isem = (isem0, isem1, isem2, isem3)
        ssem = (ssem0, ssem1)

        @pl.loop(0, ZR)
        def _(r):
            for q in range(D // 16):
                zbuf[r, pl.ds(16 * q, 16)] = jnp.zeros((16,), jnp.float32)

        for kk in range(RPT // ZR):
            pltpu.sync_copy(zbuf, acc.at[pl.ds(base + kk * ZR, ZR)])

        @pl.when(last)
        def _():
            pltpu.sync_copy(zbuf.at[pl.ds(0, TAIL)],
                            acc.at[pl.ds(NT * RPT, TAIL)])

        plsc.subcore_barrier()

        w = c * NT + s
        n = (NCHUNK - w + NW - 1) // NW

        def start_idx(i, si):
            e0 = (w + i * NW) * CH
            pltpu.async_copy(ec_hbm.at[pl.ds(e0, CH)], cb[si], isem[si])
            pltpu.async_copy(er_hbm.at[pl.ds(e0, CH)], rb[si], isem[si])
            pltpu.async_copy(ew_hbm.at[pl.ds(e0, CH)], wb[si], isem[si])

        def wait_idx(si):
            pltpu.make_async_copy(ec_hbm.at[pl.ds(0, CH)], cb[si], isem[si]).wait()
            pltpu.make_async_copy(er_hbm.at[pl.ds(0, CH)], rb[si], isem[si]).wait()
            pltpu.make_async_copy(ew_hbm.at[pl.ds(0, CH)], wb[si], isem[si]).wait()

        def wait_scatter(p):
            pltpu.make_async_copy(gb[p], acc.at[pl.ds(0, CH)], ssem[p]).wait()

        def process(j, p, si):
            q = 1 - p
            s1 = (si + 1) % 4
            s3 = (si + 3) % 4

            @pl.when(j + 1 < n)
            def _():
                wait_idx(s1)

                @pl.when(j >= 1)
                def _():
                    wait_scatter(q)

                pltpu.async_copy(h_hbm.at[cb[s1]], gb[q], gsem[q])

            pltpu.make_async_copy(h_hbm.at[pl.ds(0, CH)], gb[p], gsem[p]).wait()

            @pl.loop(0, CH, step=16)
            def _(b):
                wvec = wb[si][pl.ds(pl.multiple_of(b, 16), 16)]
                for i in range(16):
                    wv = lax.gather(
                        wvec, jnp.full((16, 1), i, jnp.int32),
                        lax.GatherDimensionNumbers(
                            offset_dims=(), collapsed_slice_dims=(0,),
                            start_index_map=(0,)),
                        slice_sizes=(1,),
                        mode=lax.GatherScatterMode.PROMISE_IN_BOUNDS)
                    for qq in range(D // 16):
                        sl = pl.ds(16 * qq, 16)
                        gb[p][b + i, sl] = gb[p][b + i, sl] * wv

            pltpu.async_copy(gb[p], acc.at[rb[si]], ssem[p], add=True)

            @pl.when(j + 3 < n)
            def _():
                start_idx(j + 3, s3)

        start_idx(0, 0)
        start_idx(1, 1)
        start_idx(2, 2)
        wait_idx(0)
        pltpu.async_copy(h_hbm.at[cb[0]], gb[0], gsem[0])

        def chunk_body(j, carry):
            for jm in range(4):
                @pl.when(j % 4 == jm)
                def _():
                    process(j, jm % 2, jm)

            return carry

        lax.fori_loop(0, n, chunk_body, 0)
        wait_scatter(0)
        wait_scatter(1)
        plsc.subcore_barrier()
        pltpu.sync_copy(acc.at[pl.ds(base, RPT)],
                        out_hbm.at[c, pl.ds(base, RPT)])

        @pl.when(last)
        def _():
            pltpu.sync_copy(acc.at[pl.ds(NT * RPT, TAIL)],
                            out_hbm.at[c, pl.ds(NT * RPT, TAIL)])

    return k(h, rows_i, cols_i, edge_weight)


def _combine_relu(p):
    BM = 2000

    def body(p_ref, o_ref):
        o_ref[...] = jnp.maximum(p_ref[0] + p_ref[1], 0.0)

    return pl.pallas_call(
        body,
        grid=(N // BM,),
        in_specs=[pl.BlockSpec((NC, BM, D), lambda i: (0, i, 0))],
        out_specs=pl.BlockSpec((BM, D), lambda i: (i, 0)),
        out_shape=jax.ShapeDtypeStruct((N, D), jnp.float32),
    )(p)


def kernel(edge_index, edge_weight, x, W):
    h = _mm(x, W)
    p = _sc_spmm(h, edge_index[0], edge_index[1], edge_weight)
    return _combine_relu(p)


# speedup vs baseline: 11.9020x; 1.0198x over previous
import dataclasses
import functools

import jax
import jax.numpy as jnp
from jax import lax
from jax.experimental import pallas as pl
from jax.experimental.pallas import tpu as pltpu
from jax.experimental.pallas import tpu_sc as plsc

N = 10000
E = 320000
D = 128
NT = 16
NC = 2
NW = NC * NT
RPT = 624
TAIL = N - NT * RPT
CH = 128
NCHUNK = E // CH
ZR = 104


def _mm(x, W):
    BM = 2000

    def body(x_ref, w_ref, o_ref):
        o_ref[...] = jnp.dot(x_ref[...], w_ref[...],
                             preferred_element_type=jnp.float32,
                             precision=jax.lax.Precision.HIGHEST)

    return pl.pallas_call(
        body,
        grid=(N // BM,),
        in_specs=[pl.BlockSpec((BM, D), lambda i: (i, 0)),
                  pl.BlockSpec((D, D), lambda i: (0, 0))],
        out_specs=pl.BlockSpec((BM, D), lambda i: (i, 0)),
        out_shape=jax.ShapeDtypeStruct((N, D), jnp.float32),
    )(x, W)


def _sc_spmm(h, packed):
    mesh = plsc.VectorSubcoreMesh(core_axis_name="c", subcore_axis_name="s")
    cparams = pltpu.CompilerParams()
    if "needs_layout_passes" in pltpu.CompilerParams.__dataclass_fields__:
        cparams = dataclasses.replace(cparams, needs_layout_passes=False)

    @functools.partial(
        pl.kernel,
        out_type=jax.ShapeDtypeStruct((NC, N, D), jnp.float32),
        mesh=mesh,
        compiler_params=cparams,
        scratch_types=(
            [pltpu.VMEM_SHARED((N, D), jnp.float32)]
            + [pltpu.VMEM((CH, D), jnp.float32)] * 2
            + [pltpu.VMEM((3, CH), jnp.int32)] * 4
            + [pltpu.VMEM((ZR, D), jnp.float32)]
            + [pltpu.SemaphoreType.DMA] * 2
            + [pltpu.SemaphoreType.DMA] * 4
            + [pltpu.SemaphoreType.DMA] * 2
        ),
    )
    def k(h_hbm, pk_hbm, out_hbm, acc, gb0, gb1, ib0, ib1, ib2, ib3,
          zbuf, gsem0, gsem1, isem0, isem1, isem2, isem3, ssem0, ssem1):
        c = lax.axis_index("c")
        s = lax.axis_index("s")
        base = s * RPT
        last = s == NT - 1

        gb = (gb0, gb1)
        ib = (ib0, ib1, ib2, ib3)
        gsem = (gsem0, gsem1)
        isem = (isem0, isem1, isem2, isem3)
        ssem = (ssem0, ssem1)

        @pl.loop(0, ZR)
        def _(r):
            for q in range(D // 16):
                zbuf[r, pl.ds(16 * q, 16)] = jnp.zeros((16,), jnp.float32)

        for kk in range(RPT // ZR):
            pltpu.sync_copy(zbuf, acc.at[pl.ds(base + kk * ZR, ZR)])

        @pl.when(last)
        def _():
            pltpu.sync_copy(zbuf.at[pl.ds(0, TAIL)],
                            acc.at[pl.ds(NT * RPT, TAIL)])

        plsc.subcore_barrier()

        w = c * NT + s
        n = (NCHUNK - w + NW - 1) // NW

        def start_idx(i, si):
            pltpu.async_copy(pk_hbm.at[w + i * NW], ib[si], isem[si])

        def wait_idx(si):
            pltpu.make_async_copy(pk_hbm.at[0], ib[si], isem[si]).wait()

        def wait_scatter(p):
            pltpu.make_async_copy(gb[p], acc.at[pl.ds(0, CH)], ssem[p]).wait()

        def process(j, p, si):
            q = 1 - p
            s1 = (si + 1) % 4
            s3 = (si + 3) % 4

            @pl.when(j + 1 < n)
            def _():
                wait_idx(s1)

                @pl.when(j >= 1)
                def _():
                    wait_scatter(q)

                pltpu.async_copy(h_hbm.at[ib[s1].at[0]], gb[q], gsem[q])

            pltpu.make_async_copy(h_hbm.at[pl.ds(0, CH)], gb[p], gsem[p]).wait()

            @pl.loop(0, CH, step=16)
            def _(b):
                wvec = plsc.bitcast(
                    ib[si][2, pl.ds(pl.multiple_of(b, 16), 16)], jnp.float32)
                for i in range(16):
                    wv = lax.gather(
                        wvec, jnp.full((16, 1), i, jnp.int32),
                        lax.GatherDimensionNumbers(
                            offset_dims=(), collapsed_slice_dims=(0,),
                            start_index_map=(0,)),
                        slice_sizes=(1,),
                        mode=lax.GatherScatterMode.PROMISE_IN_BOUNDS)
                    for qq in range(D // 16):
                        sl = pl.ds(16 * qq, 16)
                        gb[p][b + i, sl] = gb[p][b + i, sl] * wv

            pltpu.async_copy(gb[p], acc.at[ib[si].at[1]], ssem[p], add=True)

            @pl.when(j + 3 < n)
            def _():
                start_idx(j + 3, s3)

        start_idx(0, 0)
        start_idx(1, 1)
        start_idx(2, 2)
        wait_idx(0)
        pltpu.async_copy(h_hbm.at[ib[0].at[0]], gb[0], gsem[0])

        def chunk_body(j, carry):
            for jm in range(4):
                @pl.when(j % 4 == jm)
                def _():
                    process(j, jm % 2, jm)

            return carry

        lax.fori_loop(0, n, chunk_body, 0)
        wait_scatter(0)
        wait_scatter(1)
        plsc.subcore_barrier()
        pltpu.sync_copy(acc.at[pl.ds(base, RPT)],
                        out_hbm.at[c, pl.ds(base, RPT)])

        @pl.when(last)
        def _():
            pltpu.sync_copy(acc.at[pl.ds(NT * RPT, TAIL)],
                            out_hbm.at[c, pl.ds(NT * RPT, TAIL)])

    return k(h, packed)


def _combine_relu(p):
    BM = 2000

    def body(p_ref, o_ref):
        o_ref[...] = jnp.maximum(p_ref[0] + p_ref[1], 0.0)

    return pl.pallas_call(
        body,
        grid=(N // BM,),
        in_specs=[pl.BlockSpec((NC, BM, D), lambda i: (0, i, 0))],
        out_specs=pl.BlockSpec((BM, D), lambda i: (i, 0)),
        out_shape=jax.ShapeDtypeStruct((N, D), jnp.float32),
    )(p)


def kernel(edge_index, edge_weight, x, W):
    h = _mm(x, W)
    packed = jnp.stack(
        [edge_index[1].reshape(NCHUNK, CH),
         edge_index[0].reshape(NCHUNK, CH),
         lax.bitcast_convert_type(edge_weight, jnp.int32).reshape(NCHUNK, CH)],
        axis=1)
    p = _sc_spmm(h, packed)
    return _combine_relu(p)
